# baseline (speedup 1.0000x reference)
import functools

import jax
import jax.numpy as jnp
from jax import lax
from jax.experimental import pallas as pl
from jax.experimental.pallas import tpu as pltpu
from jax.experimental.pallas import tpu_sc as plsc

N_FIELDS = 26
VOCAB1 = 100001
EMB = 16
B = 16384
NC = 2
NS = 16
NW = NC * NS

ROWS = N_FIELDS * EMB
RPW = ROWS // NW
OUT_CH = 4096


def _sc_gather_t(x_t, tabs_t):
    mesh = plsc.VectorSubcoreMesh(core_axis_name="c", subcore_axis_name="s",
                                  num_cores=NC, num_subcores=NS)

    @functools.partial(
        pl.kernel,
        out_type=jax.ShapeDtypeStruct((N_FIELDS, EMB, B), jnp.float32),
        mesh=mesh,
        scratch_types=[
            pltpu.VMEM((VOCAB1,), jnp.float32),
            pltpu.VMEM((B,), jnp.int32),
            pltpu.VMEM((2, OUT_CH), jnp.float32),
            pltpu.SemaphoreType.DMA,
            pltpu.SemaphoreType.DMA((2,)),
        ],
        compiler_params=pltpu.CompilerParams(use_tc_tiling_on_sc=True,
                                             needs_layout_passes=False),
    )
    def gather_kernel(x_hbm, tab_hbm, out_hbm, row_v, idx_v, out_v,
                      row_sem, fl_sems):
        wid = lax.axis_index("s") * NC + lax.axis_index("c")
        base = wid * RPW
        flushes = [None, None]
        for r in range(RPW):
            fe = base + r
            f = fe // EMB
            e = fe - f * EMB
            if r == 0:
                need_idx = f >= 0
            else:
                need_idx = ((base + r - 1) // EMB) != f

            row_cp = pltpu.async_copy(tab_hbm.at[f, e], row_v, row_sem)

            @pl.when(need_idx)
            def _():
                pltpu.sync_copy(x_hbm.at[f], idx_v)

            row_cp.wait()
            for h in range(B // OUT_CH):
                buf = h % 2
                if flushes[buf] is not None:
                    flushes[buf].wait()

                @plsc.parallel_loop(0, OUT_CH // 16, unroll=16)
                def _(i, h=h, buf=buf):
                    ids = idx_v[pl.ds(h * OUT_CH + i * 16, 16)]
                    out_v[buf, pl.ds(i * 16, 16)] = plsc.load_gather(
                        row_v, [ids])

                flushes[buf] = pltpu.async_copy(
                    out_v.at[buf],
                    out_hbm.at[f, e, pl.ds(h * OUT_CH, OUT_CH)],
                    fl_sems.at[buf])
        for d in flushes:
            if d is not None:
                d.wait()

    return gather_kernel(x_t, tabs_t)


def _tc_mlp_t(g, z_t, w1a, w1b, b1c, w2, b2c, w3, b3c):
    bn = 2048
    dn = (((0,), (0,)), ((), ()))

    def body(g_ref, z_ref, w1a_ref, w1b_ref, b1_ref, w2_ref, b2_ref,
             w3_ref, b3_ref, o_ref):
        h = lax.dot_general(w1a_ref[...], g_ref[...], dn,
                            preferred_element_type=jnp.float32)
        h = h + lax.dot_general(w1b_ref[...], z_ref[...], dn,
                                preferred_element_type=jnp.float32)
        h = jnp.maximum(h + b1_ref[...], 0.0)
        h = lax.dot_general(w2_ref[...], h, dn,
                            preferred_element_type=jnp.float32)
        h = jnp.maximum(h + b2_ref[...], 0.0)
        o_ref[...] = (lax.dot_general(w3_ref[...], h, dn,
                                      preferred_element_type=jnp.float32)
                      + b3_ref[...])

    d_emb = ROWS
    return pl.pallas_call(
        body,
        grid=(B // bn,),
        in_specs=[
            pl.BlockSpec((d_emb, bn), lambda j: (0, j)),
            pl.BlockSpec((3, bn), lambda j: (0, j)),
            pl.BlockSpec((d_emb, 128), lambda j: (0, 0)),
            pl.BlockSpec((3, 128), lambda j: (0, 0)),
            pl.BlockSpec((128, 1), lambda j: (0, 0)),
            pl.BlockSpec((128, 64), lambda j: (0, 0)),
            pl.BlockSpec((64, 1), lambda j: (0, 0)),
            pl.BlockSpec((64, 1), lambda j: (0, 0)),
            pl.BlockSpec((1, 1), lambda j: (0, 0)),
        ],
        out_specs=pl.BlockSpec((1, bn), lambda j: (0, j)),
        out_shape=jax.ShapeDtypeStruct((1, B), jnp.float32),
        compiler_params=pltpu.CompilerParams(
            dimension_semantics=("parallel",)),
    )(g, z_t, w1a, w1b, b1c, w2, b2c, w3, b3c)


def kernel(x, z, tables, W1, b1, W2, b2, W3, b3):
    tabs_t = jnp.transpose(tables, (0, 2, 1))
    x_t = x.T
    g = _sc_gather_t(x_t, tabs_t).reshape(ROWS, B)
    y_t = _tc_mlp_t(g, z.T, W1[:ROWS], W1[ROWS:], b1.reshape(128, 1),
                    W2, b2.reshape(64, 1), W3, b3.reshape(1, 1))
    return y_t.reshape(B, 1)


# speedup vs baseline: 1.0226x; 1.0226x over previous
import functools

import jax
import jax.numpy as jnp
from jax import lax
from jax.experimental import pallas as pl
from jax.experimental.pallas import tpu as pltpu
from jax.experimental.pallas import tpu_sc as plsc

N_FIELDS = 26
VOCAB1 = 100001
EMB = 16
B = 16384
NC = 2
NS = 16
NW = NC * NS

ROWS = N_FIELDS * EMB
RPW = ROWS // NW
OUT_CH = 4096


def _sc_gather_t(x_t, tabs_t):
    mesh = plsc.VectorSubcoreMesh(core_axis_name="c", subcore_axis_name="s",
                                  num_cores=NC, num_subcores=NS)

    @functools.partial(
        pl.kernel,
        out_type=jax.ShapeDtypeStruct((N_FIELDS, EMB, B), jnp.float32),
        mesh=mesh,
        scratch_types=[
            pltpu.VMEM((VOCAB1,), jnp.float32),
            pltpu.VMEM((B,), jnp.int32),
            pltpu.VMEM((2, OUT_CH), jnp.float32),
            pltpu.SemaphoreType.DMA,
            pltpu.SemaphoreType.DMA((2,)),
        ],
        compiler_params=pltpu.CompilerParams(use_tc_tiling_on_sc=True,
                                             needs_layout_passes=False),
    )
    def gather_kernel(x_hbm, tab_hbm, out_hbm, row_v, idx_v, out_v,
                      row_sem, fl_sems):
        wid = lax.axis_index("s") * NC + lax.axis_index("c")
        base = wid * RPW
        flushes = [None, None]
        for r in range(RPW):
            fe = base + r
            f = fe // EMB
            e = fe - f * EMB
            if r == 0:
                need_idx = f >= 0
            else:
                need_idx = ((base + r - 1) // EMB) != f

            row_cp = pltpu.async_copy(tab_hbm.at[f, e], row_v, row_sem)

            @pl.when(need_idx)
            def _():
                pltpu.sync_copy(x_hbm.at[f], idx_v)

            row_cp.wait()
            for h in range(B // OUT_CH):
                buf = h % 2
                if flushes[buf] is not None:
                    flushes[buf].wait()

                @plsc.parallel_loop(0, OUT_CH // 16, unroll=8)
                def _(i, h=h, buf=buf):
                    ids = idx_v[pl.ds(h * OUT_CH + i * 16, 16)]
                    out_v[buf, pl.ds(i * 16, 16)] = plsc.load_gather(
                        row_v, [ids])

                flushes[buf] = pltpu.async_copy(
                    out_v.at[buf],
                    out_hbm.at[f, e, pl.ds(h * OUT_CH, OUT_CH)],
                    fl_sems.at[buf])
        for d in flushes:
            if d is not None:
                d.wait()

    return gather_kernel(x_t, tabs_t)


def _tc_mlp_t(g, z_t, w1a, w1b, b1c, w2, b2c, w3, b3c):
    bn = 2048
    dn = (((0,), (0,)), ((), ()))

    def body(g_ref, z_ref, w1a_ref, w1b_ref, b1_ref, w2_ref, b2_ref,
             w3_ref, b3_ref, o_ref):
        h = lax.dot_general(w1a_ref[...], g_ref[...], dn,
                            preferred_element_type=jnp.float32)
        h = h + lax.dot_general(w1b_ref[...], z_ref[...], dn,
                                preferred_element_type=jnp.float32)
        h = jnp.maximum(h + b1_ref[...], 0.0)
        h = lax.dot_general(w2_ref[...], h, dn,
                            preferred_element_type=jnp.float32)
        h = jnp.maximum(h + b2_ref[...], 0.0)
        o_ref[...] = (lax.dot_general(w3_ref[...], h, dn,
                                      preferred_element_type=jnp.float32)
                      + b3_ref[...])

    d_emb = ROWS
    return pl.pallas_call(
        body,
        grid=(B // bn,),
        in_specs=[
            pl.BlockSpec((d_emb, bn), lambda j: (0, j)),
            pl.BlockSpec((3, bn), lambda j: (0, j)),
            pl.BlockSpec((d_emb, 128), lambda j: (0, 0)),
            pl.BlockSpec((3, 128), lambda j: (0, 0)),
            pl.BlockSpec((128, 1), lambda j: (0, 0)),
            pl.BlockSpec((128, 64), lambda j: (0, 0)),
            pl.BlockSpec((64, 1), lambda j: (0, 0)),
            pl.BlockSpec((64, 1), lambda j: (0, 0)),
            pl.BlockSpec((1, 1), lambda j: (0, 0)),
        ],
        out_specs=pl.BlockSpec((1, bn), lambda j: (0, j)),
        out_shape=jax.ShapeDtypeStruct((1, B), jnp.float32),
        compiler_params=pltpu.CompilerParams(
            dimension_semantics=("parallel",)),
    )(g, z_t, w1a, w1b, b1c, w2, b2c, w3, b3c)


def kernel(x, z, tables, W1, b1, W2, b2, W3, b3):
    tabs_t = jnp.transpose(tables, (0, 2, 1))
    x_t = x.T
    g = _sc_gather_t(x_t, tabs_t).reshape(ROWS, B)
    y_t = _tc_mlp_t(g, z.T, W1[:ROWS], W1[ROWS:], b1.reshape(128, 1),
                    W2, b2.reshape(64, 1), W3, b3.reshape(1, 1))
    return y_t.reshape(B, 1)
